# baseline (speedup 1.0000x reference)
import jax
import jax.numpy as jnp
from jax import lax
from jax.experimental import pallas as pl
from jax.experimental.pallas import tpu as pltpu
from jax.experimental.pallas import tpu_sc as plsc

N_NODES = 10000
N_EDGES = 320000
D = 128
NUM_EDGE_TYPE = 64

NC = 2
NS = 16
NW = NC * NS
EW = N_EDGES // NW
C = 80
NCH = EW // C
LANES = 16
VPD = D // LANES


def _body(isrc_hbm, idst_hbm, iet_hbm, x_hbm, w_hbm, out_hbm,
          isrc_v, idst_v, iet_v, w_v, srcb, dstb, outb, sem0, sem1):
    cid = lax.axis_index("c")
    sid = lax.axis_index("s")
    wid = sid * NC + cid

    pltpu.sync_copy(w_hbm, w_v)
    pltpu.sync_copy(isrc_hbm.at[wid], isrc_v)
    pltpu.sync_copy(idst_hbm.at[wid], idst_v)
    pltpu.sync_copy(iet_hbm.at[pl.ds(wid * EW, EW + LANES)], iet_v)

    sems = (sem0, sem1)

    def fire(i, b):
        pltpu.async_copy(x_hbm.at[isrc_v.at[i]], srcb.at[b], sems[b])
        pltpu.async_copy(x_hbm.at[idst_v.at[i]], dstb.at[b], sems[b])

    def wait(i, b):
        pltpu.make_async_copy(x_hbm.at[isrc_v.at[i]], srcb.at[b], sems[b]).wait()
        pltpu.make_async_copy(x_hbm.at[idst_v.at[i]], dstb.at[b], sems[b]).wait()

    lane = lax.iota(jnp.int32, LANES)
    mask_last = lane == (LANES - 1)
    zero16 = jnp.zeros((LANES,), jnp.int32)
    offs = [lane + j * LANES for j in range(VPD)]

    def compute(i, b):
        base = i * C

        @plsc.parallel_loop(0, C, 1, unroll=8)
        def _(e):
            et_sl = iet_v[pl.ds(base + e, LANES)]
            et_b = et_sl.at[zero16].get(mode="promise_in_bounds")
            acc = None
            for j in range(VPD):
                sv = srcb[b, e, pl.ds(j * LANES, LANES)]
                dv = dstb[b, e, pl.ds(j * LANES, LANES)]
                wv = plsc.load_gather(w_v, [et_b, offs[j]])
                t = (sv * dv) * wv
                acc = t if acc is None else acc + t
            cs = plsc.cumsum(acc)
            plsc.store_compressed(outb.at[pl.ds(base + e, LANES)], cs,
                                  mask=mask_last)

    fire(0, 0)
    fire(1, 1)

    def pair(k, carry):
        i0 = k * 2
        for b in range(2):
            i = i0 + b

            @pl.when(i < NCH)
            def _():
                wait(i, b)
                compute(i, b)

                @pl.when(i + 2 < NCH)
                def _():
                    fire(i + 2, b)
        return carry

    lax.fori_loop(0, (NCH + 1) // 2, pair, 0)

    def sig(k, carry):
        v = outb[pl.ds(k * LANES, LANES)]
        outb[pl.ds(k * LANES, LANES)] = 1.0 / (1.0 + jnp.exp(-v))
        return carry

    lax.fori_loop(0, EW // LANES, sig, 0)

    pltpu.sync_copy(outb.at[pl.ds(0, EW)], out_hbm.at[pl.ds(wid * EW, EW)])


@jax.jit
def kernel(x, edge_index, edge_type, weight):
    ei = edge_index.astype(jnp.int32)
    isrc = ei[0].reshape(NW, NCH, C)
    idst = ei[1].reshape(NW, NCH, C)
    iet = jnp.concatenate(
        [edge_type.astype(jnp.int32), jnp.zeros((LANES,), jnp.int32)])

    run = pl.kernel(
        _body,
        out_type=jax.ShapeDtypeStruct((N_EDGES,), jnp.float32),
        mesh=plsc.VectorSubcoreMesh(core_axis_name="c", subcore_axis_name="s"),
        compiler_params=pltpu.CompilerParams(needs_layout_passes=False),
        scratch_types=[
            pltpu.VMEM((NCH, C), jnp.int32),
            pltpu.VMEM((NCH, C), jnp.int32),
            pltpu.VMEM((EW + LANES,), jnp.int32),
            pltpu.VMEM((NUM_EDGE_TYPE, D), jnp.float32),
            pltpu.VMEM((2, C, D), jnp.float32),
            pltpu.VMEM((2, C, D), jnp.float32),
            pltpu.VMEM((EW + LANES,), jnp.float32),
            pltpu.SemaphoreType.DMA,
            pltpu.SemaphoreType.DMA,
        ],
    )
    return run(isrc, idst, iet, x, weight)


# speedup vs baseline: 1.1064x; 1.1064x over previous
import jax
import jax.numpy as jnp
from jax import lax
from jax.experimental import pallas as pl
from jax.experimental.pallas import tpu as pltpu
from jax.experimental.pallas import tpu_sc as plsc

N_NODES = 10000
N_EDGES = 320000
D = 128
NUM_EDGE_TYPE = 64

NC = 2
NS = 16
NW = NC * NS
EW = N_EDGES // NW
C = 80
NCH = EW // C
LANES = 16
VPD = D // LANES


def _body(isrc_hbm, idst_hbm, iet_hbm, x_hbm, w_hbm, out_hbm,
          isrc_v, idst_v, iet_v, w_v, srcb, dstb, outb, sem0, sem1):
    cid = lax.axis_index("c")
    sid = lax.axis_index("s")
    wid = sid * NC + cid

    pltpu.sync_copy(w_hbm, w_v)
    pltpu.sync_copy(isrc_hbm.at[wid], isrc_v)
    pltpu.sync_copy(idst_hbm.at[wid], idst_v)
    pltpu.sync_copy(iet_hbm.at[pl.ds(wid * EW, EW + LANES)], iet_v)

    sems = (sem0, sem1)

    def fire(i, b):
        pltpu.async_copy(x_hbm.at[isrc_v.at[i]], srcb.at[b], sems[b])
        pltpu.async_copy(x_hbm.at[idst_v.at[i]], dstb.at[b], sems[b])

    def wait(i, b):
        pltpu.make_async_copy(x_hbm.at[isrc_v.at[i]], srcb.at[b], sems[b]).wait()
        pltpu.make_async_copy(x_hbm.at[idst_v.at[i]], dstb.at[b], sems[b]).wait()

    lane = lax.iota(jnp.int32, LANES)
    mask_last = lane == (LANES - 1)
    zero16 = jnp.zeros((LANES,), jnp.int32)
    offs = [lane + j * LANES for j in range(VPD)]

    def compute(i, b):
        base = i * C

        @plsc.parallel_loop(0, C, 1, unroll=2)
        def _(e):
            et_sl = iet_v[pl.ds(base + e, LANES)]
            et_b = et_sl.at[zero16].get(mode="promise_in_bounds")
            acc = None
            for j in range(VPD):
                sv = srcb[b, e, pl.ds(j * LANES, LANES)]
                dv = dstb[b, e, pl.ds(j * LANES, LANES)]
                wv = plsc.load_gather(w_v, [et_b, offs[j]])
                t = (sv * dv) * wv
                acc = t if acc is None else acc + t
            cs = plsc.cumsum(acc)
            plsc.store_compressed(outb.at[pl.ds(base + e, LANES)], cs,
                                  mask=mask_last)

    fire(0, 0)
    fire(1, 1)

    def pair(k, carry):
        i0 = k * 2
        for b in range(2):
            i = i0 + b

            @pl.when(i < NCH)
            def _():
                wait(i, b)
                compute(i, b)

                @pl.when(i + 2 < NCH)
                def _():
                    fire(i + 2, b)
        return carry

    lax.fori_loop(0, (NCH + 1) // 2, pair, 0)

    def sig(k, carry):
        v = outb[pl.ds(k * LANES, LANES)]
        outb[pl.ds(k * LANES, LANES)] = 1.0 / (1.0 + jnp.exp(-v))
        return carry

    lax.fori_loop(0, EW // LANES, sig, 0)

    pltpu.sync_copy(outb.at[pl.ds(0, EW)], out_hbm.at[pl.ds(wid * EW, EW)])


@jax.jit
def kernel(x, edge_index, edge_type, weight):
    ei = edge_index.astype(jnp.int32)
    isrc = ei[0].reshape(NW, NCH, C)
    idst = ei[1].reshape(NW, NCH, C)
    iet = jnp.concatenate(
        [edge_type.astype(jnp.int32), jnp.zeros((LANES,), jnp.int32)])

    run = pl.kernel(
        _body,
        out_type=jax.ShapeDtypeStruct((N_EDGES,), jnp.float32),
        mesh=plsc.VectorSubcoreMesh(core_axis_name="c", subcore_axis_name="s"),
        compiler_params=pltpu.CompilerParams(needs_layout_passes=False),
        scratch_types=[
            pltpu.VMEM((NCH, C), jnp.int32),
            pltpu.VMEM((NCH, C), jnp.int32),
            pltpu.VMEM((EW + LANES,), jnp.int32),
            pltpu.VMEM((NUM_EDGE_TYPE, D), jnp.float32),
            pltpu.VMEM((2, C, D), jnp.float32),
            pltpu.VMEM((2, C, D), jnp.float32),
            pltpu.VMEM((EW + LANES,), jnp.float32),
            pltpu.SemaphoreType.DMA,
            pltpu.SemaphoreType.DMA,
        ],
    )
    return run(isrc, idst, iet, x, weight)


# speedup vs baseline: 1.3148x; 1.1883x over previous
import jax
import jax.numpy as jnp
from jax import lax
from jax.experimental import pallas as pl
from jax.experimental.pallas import tpu as pltpu
from jax.experimental.pallas import tpu_sc as plsc

N_NODES = 10000
N_EDGES = 320000
D = 128
NUM_EDGE_TYPE = 64

NC = 2
NS = 16
NW = NC * NS
EW = N_EDGES // NW
C = 80
NCH = EW // C
LANES = 16
VPD = D // LANES


def _body(isrc_hbm, idst_hbm, iet_hbm, x_hbm, w_hbm, out_hbm,
          isrc_v, idst_v, iet_v, w_v, srcb, dstb, outb, sem0, sem1, sem2):
    cid = lax.axis_index("c")
    sid = lax.axis_index("s")
    wid = sid * NC + cid

    pltpu.sync_copy(w_hbm, w_v)
    pltpu.sync_copy(isrc_hbm.at[wid], isrc_v)
    pltpu.sync_copy(idst_hbm.at[wid], idst_v)
    pltpu.sync_copy(iet_hbm.at[pl.ds(wid * EW, EW + LANES)], iet_v)

    sems = (sem0, sem1, sem2)

    def fire(i, b):
        pltpu.async_copy(x_hbm.at[isrc_v.at[i]], srcb.at[b], sems[b])
        pltpu.async_copy(x_hbm.at[idst_v.at[i]], dstb.at[b], sems[b])

    def wait(i, b):
        pltpu.make_async_copy(x_hbm.at[isrc_v.at[i]], srcb.at[b], sems[b]).wait()
        pltpu.make_async_copy(x_hbm.at[idst_v.at[i]], dstb.at[b], sems[b]).wait()

    lane = lax.iota(jnp.int32, LANES)
    mask_last = lane == (LANES - 1)
    zero16 = jnp.zeros((LANES,), jnp.int32)
    offs = [lane + j * LANES for j in range(VPD)]

    def compute(i, b):
        base = i * C

        @plsc.parallel_loop(0, C, 1, unroll=2)
        def _(e):
            et_sl = iet_v[pl.ds(base + e, LANES)]
            et_b = et_sl.at[zero16].get(mode="promise_in_bounds")
            acc = None
            for j in range(VPD):
                sv = srcb[b, e, pl.ds(j * LANES, LANES)]
                dv = dstb[b, e, pl.ds(j * LANES, LANES)]
                wv = plsc.load_gather(w_v, [et_b, offs[j]])
                t = (sv * dv) * wv
                acc = t if acc is None else acc + t
            cs = plsc.cumsum(acc)
            plsc.store_compressed(outb.at[pl.ds(base + e, LANES)], cs,
                                  mask=mask_last)

    fire(0, 0)
    fire(1, 1)
    fire(2, 2)

    NB = 3

    def triple(k, carry):
        i0 = k * NB
        for b in range(NB):
            i = i0 + b

            @pl.when(i < NCH)
            def _():
                wait(i, b)
                compute(i, b)

                @pl.when(i + NB < NCH)
                def _():
                    fire(i + NB, b)
        return carry

    lax.fori_loop(0, (NCH + NB - 1) // NB, triple, 0)

    @plsc.parallel_loop(0, EW // LANES, 1, unroll=8)
    def _(k):
        v = outb[pl.ds(k * LANES, LANES)]
        outb[pl.ds(k * LANES, LANES)] = 1.0 / (1.0 + jnp.exp(-v))

    pltpu.sync_copy(outb.at[pl.ds(0, EW)], out_hbm.at[pl.ds(wid * EW, EW)])


@jax.jit
def kernel(x, edge_index, edge_type, weight):
    ei = edge_index.astype(jnp.int32)
    isrc = ei[0].reshape(NW, NCH, C)
    idst = ei[1].reshape(NW, NCH, C)
    iet = jnp.concatenate(
        [edge_type.astype(jnp.int32), jnp.zeros((LANES,), jnp.int32)])

    run = pl.kernel(
        _body,
        out_type=jax.ShapeDtypeStruct((N_EDGES,), jnp.float32),
        mesh=plsc.VectorSubcoreMesh(core_axis_name="c", subcore_axis_name="s"),
        compiler_params=pltpu.CompilerParams(needs_layout_passes=False),
        scratch_types=[
            pltpu.VMEM((NCH, C), jnp.int32),
            pltpu.VMEM((NCH, C), jnp.int32),
            pltpu.VMEM((EW + LANES,), jnp.int32),
            pltpu.VMEM((NUM_EDGE_TYPE, D), jnp.float32),
            pltpu.VMEM((3, C, D), jnp.float32),
            pltpu.VMEM((3, C, D), jnp.float32),
            pltpu.VMEM((EW + LANES,), jnp.float32),
            pltpu.SemaphoreType.DMA,
            pltpu.SemaphoreType.DMA,
            pltpu.SemaphoreType.DMA,
        ],
    )
    return run(isrc, idst, iet, x, weight)


# speedup vs baseline: 1.3175x; 1.0020x over previous
import jax
import jax.numpy as jnp
from jax import lax
from jax.experimental import pallas as pl
from jax.experimental.pallas import tpu as pltpu
from jax.experimental.pallas import tpu_sc as plsc

N_NODES = 10000
N_EDGES = 320000
D = 128
NUM_EDGE_TYPE = 64

NC = 2
NS = 16
NW = NC * NS
EW = N_EDGES // NW
C = 80
NCH = EW // C
LANES = 16
VPD = D // LANES


def _body(isrc_hbm, idst_hbm, iet_hbm, x_hbm, w_hbm, out_hbm,
          isrc_v, idst_v, iet_v, w_v, srcb, dstb, outb, sem0, sem1, sem2):
    cid = lax.axis_index("c")
    sid = lax.axis_index("s")
    wid = sid * NC + cid

    pltpu.sync_copy(w_hbm, w_v)
    pltpu.sync_copy(isrc_hbm.at[wid], isrc_v)
    pltpu.sync_copy(idst_hbm.at[wid], idst_v)
    pltpu.sync_copy(iet_hbm.at[pl.ds(wid * EW, EW + LANES)], iet_v)

    sems = (sem0, sem1, sem2)

    def fire(i, b):
        pltpu.async_copy(x_hbm.at[isrc_v.at[i]], srcb.at[b], sems[b])
        pltpu.async_copy(x_hbm.at[idst_v.at[i]], dstb.at[b], sems[b])

    def wait(i, b):
        pltpu.make_async_copy(x_hbm.at[isrc_v.at[i]], srcb.at[b], sems[b]).wait()
        pltpu.make_async_copy(x_hbm.at[idst_v.at[i]], dstb.at[b], sems[b]).wait()

    lane = lax.iota(jnp.int32, LANES)
    mask_last = lane == (LANES - 1)
    zero16 = jnp.zeros((LANES,), jnp.int32)
    offs = [lane + j * LANES for j in range(VPD)]

    def compute(i, b):
        base = i * C

        @plsc.parallel_loop(0, C, 1, unroll=2)
        def _(e):
            et_sl = iet_v[pl.ds(base + e, LANES)]
            et_b = et_sl.at[zero16].get(mode="promise_in_bounds")
            acc = [None, None]
            for j in range(VPD):
                sv = srcb[b, e, pl.ds(j * LANES, LANES)]
                dv = dstb[b, e, pl.ds(j * LANES, LANES)]
                wv = plsc.load_gather(w_v, [et_b, offs[j]])
                t = (sv * dv) * wv
                p = j % 2
                acc[p] = t if acc[p] is None else acc[p] + t
            cs = plsc.cumsum(acc[0] + acc[1])
            plsc.store_compressed(outb.at[pl.ds(base + e, LANES)], cs,
                                  mask=mask_last)

    fire(0, 0)
    fire(1, 1)
    fire(2, 2)

    NB = 3

    def triple(k, carry):
        i0 = k * NB
        for b in range(NB):
            i = i0 + b

            @pl.when(i < NCH)
            def _():
                wait(i, b)
                compute(i, b)

                @pl.when(i + NB < NCH)
                def _():
                    fire(i + NB, b)
        return carry

    lax.fori_loop(0, (NCH + NB - 1) // NB, triple, 0)

    @plsc.parallel_loop(0, EW // LANES, 1, unroll=8)
    def _(k):
        v = outb[pl.ds(k * LANES, LANES)]
        outb[pl.ds(k * LANES, LANES)] = 1.0 / (1.0 + jnp.exp(-v))

    pltpu.sync_copy(outb.at[pl.ds(0, EW)], out_hbm.at[pl.ds(wid * EW, EW)])


@jax.jit
def kernel(x, edge_index, edge_type, weight):
    ei = edge_index.astype(jnp.int32)
    isrc = ei[0].reshape(NW, NCH, C)
    idst = ei[1].reshape(NW, NCH, C)
    iet = jnp.concatenate(
        [edge_type.astype(jnp.int32), jnp.zeros((LANES,), jnp.int32)])

    run = pl.kernel(
        _body,
        out_type=jax.ShapeDtypeStruct((N_EDGES,), jnp.float32),
        mesh=plsc.VectorSubcoreMesh(core_axis_name="c", subcore_axis_name="s"),
        compiler_params=pltpu.CompilerParams(needs_layout_passes=False),
        scratch_types=[
            pltpu.VMEM((NCH, C), jnp.int32),
            pltpu.VMEM((NCH, C), jnp.int32),
            pltpu.VMEM((EW + LANES,), jnp.int32),
            pltpu.VMEM((NUM_EDGE_TYPE, D), jnp.float32),
            pltpu.VMEM((3, C, D), jnp.float32),
            pltpu.VMEM((3, C, D), jnp.float32),
            pltpu.VMEM((EW + LANES,), jnp.float32),
            pltpu.SemaphoreType.DMA,
            pltpu.SemaphoreType.DMA,
            pltpu.SemaphoreType.DMA,
        ],
    )
    return run(isrc, idst, iet, x, weight)
